# Initial kernel scaffold; baseline (speedup 1.0000x reference)
import jax
import jax.numpy as jnp
from jax.experimental import pallas as pl


def kernel(x, edge_index, edge_attr, new_edge_index, cayley_g, cayley_attr, max_node, atom_emb, bond_emb, W1, b1, g1, be1, W2, b2, eps, bn_g, bn_b):
    raise NotImplementedError("write your pallas kernel here")


# speedup vs baseline: 3.3651x; 3.3651x over previous
import functools

import jax
import jax.numpy as jnp
from jax import lax
from jax.experimental import pallas as pl
from jax.experimental.pallas import tpu as pltpu
from jax.experimental.pallas import tpu_sc as plsc

D = 128
NC = 2
NS = 16
NW = NC * NS
K = 128


def _round_up(a, b):
    return (a + b - 1) // b * b


SCHUNK = 1024


def _sc_gather_scatter(n_agg, n_edges, with_ee):
    if with_ee:
        K, NBUF = 64, 1
    else:
        K, NBUF = 128, 2
    SUB = SCHUNK // K
    per_w = n_edges // NW
    n_super = per_w // SCHUNK
    rps = n_agg // NS

    mesh = plsc.VectorSubcoreMesh(core_axis_name="c", subcore_axis_name="s")

    scratch = [
        pltpu.VMEM((SUB, K), jnp.int32),
        pltpu.VMEM((SUB, K), jnp.int32),
        pltpu.VMEM((SUB, K), jnp.int32),
        pltpu.VMEM((NBUF, K, D), jnp.float32),
        pltpu.VMEM((64 * D,), jnp.float32),
        pltpu.VMEM_SHARED((n_agg, D), jnp.float32),
        pltpu.SemaphoreType.DMA,
    ]
    if not with_ee:
        scratch[2] = pltpu.VMEM((8,), jnp.int32)
        scratch[4] = pltpu.VMEM((16,), jnp.float32)

    def body(table_hbm, src_hbm, dst_hbm, code_hbm, ee_hbm, out_hbm,
             src_m, dst_m, code_m, rows_v, ee_v, agg_sh, sem):
        c = lax.axis_index("c")
        s = lax.axis_index("s")
        wid = s * NC + c

        zero16 = jnp.zeros((16,), jnp.float32)
        rv0 = rows_v.at[0]

        def zfill(i, carry):
            for j in range(D // 16):
                rv0[i, pl.ds(j * 16, 16)] = zero16
            return carry

        lax.fori_loop(0, K, zfill, 0)
        row0 = s * rps
        for t in range(rps // K):
            pltpu.sync_copy(rows_v.at[0], agg_sh.at[pl.ds(row0 + t * K, K)])
        rem = rps % K
        if rem:
            pltpu.sync_copy(rows_v.at[0, pl.ds(0, rem)],
                            agg_sh.at[pl.ds(row0 + (rps // K) * K, rem)])
        if with_ee:
            pltpu.sync_copy(ee_hbm, ee_v)
        plsc.subcore_barrier()

        crow0 = wid * (per_w // K)
        lanes = lax.iota(jnp.int32, 16)

        def superchunk(t, carry):
            cbase = crow0 + t * SUB
            pltpu.sync_copy(src_hbm.at[pl.ds(cbase, SUB)], src_m)
            pltpu.sync_copy(dst_hbm.at[pl.ds(cbase, SUB)], dst_m)
            if with_ee:
                pltpu.sync_copy(code_hbm.at[pl.ds(cbase, SUB)], code_m)

            if NBUF == 1:
                for j in range(SUB):
                    pltpu.async_copy(table_hbm.at[src_m.at[j]],
                                     rows_v.at[0], sem).wait()
                    if with_ee:
                        def msg16(g, cr):
                            codes = code_m[j, pl.ds(g * 16, 16)]

                            def one(e, cr2):
                                cb = jnp.take_along_axis(
                                    codes, jnp.full((16,), e, jnp.int32),
                                    axis=0)
                                ee_base = cb * D + lanes
                                row = g * 16 + e
                                for q in range(D // 16):
                                    sl = pl.ds(q * 16, 16)
                                    eev = plsc.load_gather(
                                        ee_v, [ee_base + q * 16])
                                    rv0[row, sl] = jnp.maximum(
                                        rv0[row, sl] + eev, 0.0)
                                return cr2

                            lax.fori_loop(0, 16, one, 0)
                            return cr

                        lax.fori_loop(0, K // 16, msg16, 0)
                    pltpu.sync_copy(rows_v.at[0], agg_sh.at[dst_m.at[j]],
                                    add=True)
            else:
                descs = [None] * SUB
                descs[0] = pltpu.async_copy(table_hbm.at[src_m.at[0]],
                                            rows_v.at[0], sem)
                for j in range(SUB):
                    descs[j].wait()
                    if j + 1 < SUB:
                        descs[j + 1] = pltpu.async_copy(
                            table_hbm.at[src_m.at[j + 1]],
                            rows_v.at[(j + 1) % 2], sem)
                    pltpu.sync_copy(rows_v.at[j % 2],
                                    agg_sh.at[dst_m.at[j]], add=True)
            return carry

        lax.fori_loop(0, n_super, superchunk, 0)
        plsc.subcore_barrier()

        for t in range(rps // K):
            pltpu.sync_copy(agg_sh.at[pl.ds(row0 + t * K, K)],
                            out_hbm.at[c, pl.ds(row0 + t * K, K)])
        if rem:
            pltpu.sync_copy(agg_sh.at[pl.ds(row0 + (rps // K) * K, rem)],
                            out_hbm.at[c, pl.ds(row0 + (rps // K) * K, rem)])

    return pl.kernel(
        body,
        out_type=jax.ShapeDtypeStruct((NC, n_agg, D), jnp.float32),
        mesh=mesh,
        scratch_types=scratch,
        compiler_params=pltpu.CompilerParams(needs_layout_passes=False),
    ), K


def _tc_prep(N, M):
    def body(p_ref, bond_ref, h_ref, ee_ref):
        h0 = p_ref[0, :N, :] + p_ref[1, :N, :]
        h_ref[:N, :] = h0
        avg = jnp.mean(h0, axis=0, keepdims=True)
        h_ref[N:, :] = jnp.broadcast_to(avg, (M - N, D))
        b0 = bond_ref[:, 0]
        b1 = bond_ref[:, 1]
        b2 = bond_ref[:, 2]
        ee = (b0[:, :, None, None, :] + b1[:, None, :, None, :]
              + b2[:, None, None, :, :])
        Lw = ee.shape[0]
        ee_ref[...] = ee.reshape(Lw, 64, D)

    def run(parts, bond_emb):
        Lw = bond_emb.shape[0]
        return pl.pallas_call(
            body,
            out_shape=(jax.ShapeDtypeStruct((M, D), jnp.float32),
                       jax.ShapeDtypeStruct((Lw, 64, D), jnp.float32)),
        )(parts, bond_emb)

    return run


def _tc_mlp1(M):
    def body(h_ref, p_ref, W1_ref, b1_ref, eps_ref, z1_o, m_o):
        agg = p_ref[0, :M, :] + p_ref[1, :M, :]
        z = (1.0 + eps_ref[0, 0]) * h_ref[...] + agg
        z1 = jnp.dot(z, W1_ref[...], preferred_element_type=jnp.float32)
        z1 = z1 + b1_ref[...]
        z1_o[...] = z1
        m_o[...] = jnp.mean(z1, axis=0, keepdims=True)

    def run(h, parts, W1l, b1l, epsl):
        return pl.pallas_call(
            body,
            out_shape=(jax.ShapeDtypeStruct((M, 2 * D), jnp.float32),
                       jax.ShapeDtypeStruct((1, 2 * D), jnp.float32)),
        )(h, parts, W1l, b1l.reshape(1, -1), epsl.reshape(1, 1))

    return run


def _tc_mlp2(M):
    def body(z1d_ref, g_ref, be_ref, W2_ref, b2_ref, z2_o, m_o):
        z1n = z1d_ref[...] * g_ref[...] + be_ref[...]
        z1r = jnp.maximum(z1n, 0.0)
        z2 = jnp.dot(z1r, W2_ref[...], preferred_element_type=jnp.float32)
        z2 = z2 + b2_ref[...]
        z2_o[...] = z2
        m_o[...] = jnp.mean(z2, axis=0, keepdims=True)

    def run(z1d, g1l, be1l, W2l, b2l):
        return pl.pallas_call(
            body,
            out_shape=(jax.ShapeDtypeStruct((M, D), jnp.float32),
                       jax.ShapeDtypeStruct((1, D), jnp.float32)),
        )(z1d, g1l.reshape(1, -1), be1l.reshape(1, -1), W2l,
          b2l.reshape(1, -1))

    return run


def _tc_bnout(M, last):
    def body(z2d_ref, g_ref, b_ref, o_ref):
        z2n = z2d_ref[...] * g_ref[...] + b_ref[...]
        if not last:
            z2n = jnp.maximum(z2n, 0.0)
        o_ref[...] = z2n

    def run(z2d, gl, bl):
        return pl.pallas_call(
            body, out_shape=jax.ShapeDtypeStruct((M, D), jnp.float32),
        )(z2d, gl.reshape(1, -1), bl.reshape(1, -1))

    return run


def _pad_edges(src, dst, code, trash):
    n = src.shape[0]
    npad = _round_up(n, NW * SCHUNK)
    pad = npad - n
    if pad:
        src = jnp.concatenate([src, jnp.zeros((pad,), jnp.int32)])
        dst = jnp.concatenate([dst, jnp.full((pad,), trash, jnp.int32)])
        code = jnp.concatenate([code, jnp.zeros((pad,), jnp.int32)])
    return src, dst, code, npad


def kernel(x, edge_index, edge_attr, new_edge_index, cayley_g, cayley_attr,
           max_node, atom_emb, bond_emb, W1, b1, g1, be1, W2, b2, eps,
           bn_g, bn_b):
    N = x.shape[0]
    E = edge_attr.shape[0]
    E2 = new_edge_index.shape[1]
    C = cayley_g.shape[1]
    M = E2 - E + N
    L = W1.shape[0]

    Npad = _round_up(N + 1, K)
    Mpad = _round_up(M + 1, K)

    x32 = x.astype(jnp.int32)
    atom_flat = atom_emb.reshape(9 * atom_emb.shape[1], D)
    emb_src = (x32 + jnp.arange(9, dtype=jnp.int32)[None, :]
               * atom_emb.shape[1]).reshape(-1)
    emb_dst = jnp.repeat(jnp.arange(N, dtype=jnp.int32), 9)
    zero_codes = jnp.zeros_like(emb_src)
    emb_src, emb_dst, emb_code, _ = _pad_edges(emb_src, emb_dst, zero_codes,
                                               Npad - 1)

    ea = edge_attr.astype(jnp.int32)
    codes_e = ea[:, 0] * 16 + ea[:, 1] * 4 + ea[:, 2]
    codes_even = jnp.concatenate(
        [codes_e, jnp.zeros((E2 - E,), jnp.int32)])
    src_even, dst_even, code_even, _ = _pad_edges(
        new_edge_index[0].astype(jnp.int32),
        new_edge_index[1].astype(jnp.int32), codes_even, Mpad - 1)
    src_odd, dst_odd, code_odd, _ = _pad_edges(
        cayley_g[0].astype(jnp.int32), cayley_g[1].astype(jnp.int32),
        jnp.zeros((C,), jnp.int32), Mpad - 1)

    dummy_ee = jnp.zeros((64 * D,), jnp.float32)

    sc_embed, k_emb = _sc_gather_scatter(Npad, emb_src.shape[0],
                                         with_ee=False)
    parts0 = sc_embed(atom_flat, emb_src.reshape(-1, k_emb),
                      emb_dst.reshape(-1, k_emb),
                      emb_code.reshape(-1, k_emb), dummy_ee)

    h, ee_all = _tc_prep(N, M)(parts0, bond_emb)

    sc_even, k_ev = _sc_gather_scatter(Mpad, src_even.shape[0], with_ee=True)
    sc_odd, k_od = _sc_gather_scatter(Mpad, src_odd.shape[0], with_ee=True)
    src_even, dst_even, code_even = (a.reshape(-1, k_ev) for a in
                                     (src_even, dst_even, code_even))
    src_odd, dst_odd, code_odd = (a.reshape(-1, k_od) for a in
                                  (src_odd, dst_odd, code_odd))

    mlp1 = _tc_mlp1(M)
    mlp2 = _tc_mlp2(M)
    for l in range(L):
        eel = ee_all[l].reshape(-1)
        if l % 2 == 1:
            parts = sc_odd(h, src_odd, dst_odd, code_odd, eel)
        else:
            parts = sc_even(h, src_even, dst_even, code_even, eel)
        z1, m1 = mlp1(h, parts, W1[l], b1[l], eps[l])
        v1 = jnp.mean(jnp.square(z1 - m1[0]), axis=0)
        z1d = (z1 - m1[0]) / jnp.sqrt(v1 + 1e-5)
        z2, m2 = mlp2(z1d, g1[l], be1[l], W2[l], b2[l])
        v2 = jnp.mean(jnp.square(z2 - m2[0]), axis=0)
        z2d = (z2 - m2[0]) / jnp.sqrt(v2 + 1e-5)
        h = _tc_bnout(M, last=(l == L - 1))(z2d, bn_g[l], bn_b[l])
    return h


# speedup vs baseline: 4.1266x; 1.2263x over previous
import functools

import jax
import jax.numpy as jnp
from jax import lax
from jax.experimental import pallas as pl
from jax.experimental.pallas import tpu as pltpu
from jax.experimental.pallas import tpu_sc as plsc

D = 128
NC = 2
NS = 16
NW = NC * NS
K = 128


def _round_up(a, b):
    return (a + b - 1) // b * b


SCHUNK = 1024


def _sc_gather_scatter(n_agg, n_edges, with_ee):
    if with_ee:
        K, SUB = 32, 16
    else:
        K, SUB = 128, 8
    SCK = K * SUB
    per_w = n_edges // NW
    n_super = per_w // SCK
    rps = n_agg // NS

    mesh = plsc.VectorSubcoreMesh(core_axis_name="c", subcore_axis_name="s")

    scratch = [
        pltpu.VMEM((SUB, K), jnp.int32),
        pltpu.VMEM((SUB, K), jnp.int32),
        pltpu.VMEM((SUB, K), jnp.int32),
        pltpu.VMEM((2, K, D), jnp.float32),
        pltpu.VMEM((64 * D,), jnp.float32),
        pltpu.VMEM_SHARED((n_agg, D), jnp.float32),
        pltpu.SemaphoreType.DMA,
        pltpu.SemaphoreType.DMA,
    ]
    if not with_ee:
        scratch[2] = pltpu.VMEM((8,), jnp.int32)
        scratch[4] = pltpu.VMEM((16,), jnp.float32)

    def body(table_hbm, src_hbm, dst_hbm, code_hbm, ee_hbm, out_hbm,
             src_m, dst_m, code_m, rows_v, ee_v, agg_sh, sem, sem_s):
        c = lax.axis_index("c")
        s = lax.axis_index("s")
        wid = s * NC + c

        zero16 = jnp.zeros((16,), jnp.float32)
        rv0 = rows_v.at[0]

        def zfill(i, carry):
            for j in range(D // 16):
                rv0[i, pl.ds(j * 16, 16)] = zero16
            return carry

        lax.fori_loop(0, K, zfill, 0)
        row0 = s * rps
        for t in range(rps // K):
            pltpu.sync_copy(rows_v.at[0], agg_sh.at[pl.ds(row0 + t * K, K)])
        rem = rps % K
        if rem:
            pltpu.sync_copy(rows_v.at[0, pl.ds(0, rem)],
                            agg_sh.at[pl.ds(row0 + (rps // K) * K, rem)])
        if with_ee:
            pltpu.sync_copy(ee_hbm, ee_v)
        plsc.subcore_barrier()

        crow0 = wid * (per_w // K)
        lanes = lax.iota(jnp.int32, 16)

        def superchunk(t, carry):
            cbase = crow0 + t * SUB
            pltpu.sync_copy(src_hbm.at[pl.ds(cbase, SUB)], src_m)
            pltpu.sync_copy(dst_hbm.at[pl.ds(cbase, SUB)], dst_m)
            if with_ee:
                pltpu.sync_copy(code_hbm.at[pl.ds(cbase, SUB)], code_m)

            gd = [None] * SUB
            sd = [None] * SUB
            gd[0] = pltpu.async_copy(table_hbm.at[src_m.at[0]],
                                     rows_v.at[0], sem)
            for j in range(SUB):
                b = j % 2
                gd[j].wait()
                if j + 1 < SUB:
                    if j >= 1:
                        sd[j - 1].wait()
                    gd[j + 1] = pltpu.async_copy(
                        table_hbm.at[src_m.at[j + 1]],
                        rows_v.at[(j + 1) % 2], sem)
                if with_ee:
                    rvb = rows_v.at[b]

                    def msg16(g, cr, j=j, rvb=rvb):
                        codes = code_m[j, pl.ds(g * 16, 16)]

                        def one(e, cr2):
                            cb = jnp.take_along_axis(
                                codes, jnp.full((16,), e, jnp.int32),
                                axis=0)
                            ee_base = cb * D + lanes
                            row = g * 16 + e
                            for q in range(D // 16):
                                sl = pl.ds(q * 16, 16)
                                eev = plsc.load_gather(
                                    ee_v, [ee_base + q * 16])
                                rvb[row, sl] = jnp.maximum(
                                    rvb[row, sl] + eev, 0.0)
                            return cr2

                        lax.fori_loop(0, 16, one, 0)
                        return cr

                    lax.fori_loop(0, K // 16, msg16, 0)
                sd[j] = pltpu.async_copy(rows_v.at[b],
                                         agg_sh.at[dst_m.at[j]], sem_s,
                                         add=True)
            sd[SUB - 2].wait()
            sd[SUB - 1].wait()
            return carry

        lax.fori_loop(0, n_super, superchunk, 0)
        plsc.subcore_barrier()

        for t in range(rps // K):
            pltpu.sync_copy(agg_sh.at[pl.ds(row0 + t * K, K)],
                            out_hbm.at[c, pl.ds(row0 + t * K, K)])
        if rem:
            pltpu.sync_copy(agg_sh.at[pl.ds(row0 + (rps // K) * K, rem)],
                            out_hbm.at[c, pl.ds(row0 + (rps // K) * K, rem)])

    return pl.kernel(
        body,
        out_type=jax.ShapeDtypeStruct((NC, n_agg, D), jnp.float32),
        mesh=mesh,
        scratch_types=scratch,
        compiler_params=pltpu.CompilerParams(needs_layout_passes=False),
    ), K


def _tc_prep(N, M):
    def body(p_ref, bond_ref, h_ref, ee_ref):
        h0 = p_ref[0, :N, :] + p_ref[1, :N, :]
        h_ref[:N, :] = h0
        avg = jnp.mean(h0, axis=0, keepdims=True)
        h_ref[N:, :] = jnp.broadcast_to(avg, (M - N, D))
        b0 = bond_ref[:, 0]
        b1 = bond_ref[:, 1]
        b2 = bond_ref[:, 2]
        ee = (b0[:, :, None, None, :] + b1[:, None, :, None, :]
              + b2[:, None, None, :, :])
        Lw = ee.shape[0]
        ee_ref[...] = ee.reshape(Lw, 64, D)

    def run(parts, bond_emb):
        Lw = bond_emb.shape[0]
        return pl.pallas_call(
            body,
            out_shape=(jax.ShapeDtypeStruct((M, D), jnp.float32),
                       jax.ShapeDtypeStruct((Lw, 64, D), jnp.float32)),
        )(parts, bond_emb)

    return run


def _tc_mlp1(M):
    def body(h_ref, p_ref, W1_ref, b1_ref, eps_ref, z1_o, m_o):
        agg = p_ref[0, :M, :] + p_ref[1, :M, :]
        z = (1.0 + eps_ref[0, 0]) * h_ref[...] + agg
        z1 = jnp.dot(z, W1_ref[...], preferred_element_type=jnp.float32)
        z1 = z1 + b1_ref[...]
        z1_o[...] = z1
        m_o[...] = jnp.mean(z1, axis=0, keepdims=True)

    def run(h, parts, W1l, b1l, epsl):
        return pl.pallas_call(
            body,
            out_shape=(jax.ShapeDtypeStruct((M, 2 * D), jnp.float32),
                       jax.ShapeDtypeStruct((1, 2 * D), jnp.float32)),
        )(h, parts, W1l, b1l.reshape(1, -1), epsl.reshape(1, 1))

    return run


def _tc_mlp2(M):
    def body(z1d_ref, g_ref, be_ref, W2_ref, b2_ref, z2_o, m_o):
        z1n = z1d_ref[...] * g_ref[...] + be_ref[...]
        z1r = jnp.maximum(z1n, 0.0)
        z2 = jnp.dot(z1r, W2_ref[...], preferred_element_type=jnp.float32)
        z2 = z2 + b2_ref[...]
        z2_o[...] = z2
        m_o[...] = jnp.mean(z2, axis=0, keepdims=True)

    def run(z1d, g1l, be1l, W2l, b2l):
        return pl.pallas_call(
            body,
            out_shape=(jax.ShapeDtypeStruct((M, D), jnp.float32),
                       jax.ShapeDtypeStruct((1, D), jnp.float32)),
        )(z1d, g1l.reshape(1, -1), be1l.reshape(1, -1), W2l,
          b2l.reshape(1, -1))

    return run


def _tc_bnout(M, last):
    def body(z2d_ref, g_ref, b_ref, o_ref):
        z2n = z2d_ref[...] * g_ref[...] + b_ref[...]
        if not last:
            z2n = jnp.maximum(z2n, 0.0)
        o_ref[...] = z2n

    def run(z2d, gl, bl):
        return pl.pallas_call(
            body, out_shape=jax.ShapeDtypeStruct((M, D), jnp.float32),
        )(z2d, gl.reshape(1, -1), bl.reshape(1, -1))

    return run


def _pad_edges(src, dst, code, trash):
    n = src.shape[0]
    npad = _round_up(n, NW * SCHUNK)
    pad = npad - n
    if pad:
        src = jnp.concatenate([src, jnp.zeros((pad,), jnp.int32)])
        dst = jnp.concatenate([dst, jnp.full((pad,), trash, jnp.int32)])
        code = jnp.concatenate([code, jnp.zeros((pad,), jnp.int32)])
    return src, dst, code, npad


def kernel(x, edge_index, edge_attr, new_edge_index, cayley_g, cayley_attr,
           max_node, atom_emb, bond_emb, W1, b1, g1, be1, W2, b2, eps,
           bn_g, bn_b):
    N = x.shape[0]
    E = edge_attr.shape[0]
    E2 = new_edge_index.shape[1]
    C = cayley_g.shape[1]
    M = E2 - E + N
    L = W1.shape[0]

    Npad = _round_up(N + 1, K)
    Mpad = _round_up(M + 1, K)

    x32 = x.astype(jnp.int32)
    atom_flat = atom_emb.reshape(9 * atom_emb.shape[1], D)
    emb_src = (x32 + jnp.arange(9, dtype=jnp.int32)[None, :]
               * atom_emb.shape[1]).reshape(-1)
    emb_dst = jnp.repeat(jnp.arange(N, dtype=jnp.int32), 9)
    zero_codes = jnp.zeros_like(emb_src)
    emb_src, emb_dst, emb_code, _ = _pad_edges(emb_src, emb_dst, zero_codes,
                                               Npad - 1)

    ea = edge_attr.astype(jnp.int32)
    codes_e = ea[:, 0] * 16 + ea[:, 1] * 4 + ea[:, 2]
    codes_even = jnp.concatenate(
        [codes_e, jnp.zeros((E2 - E,), jnp.int32)])
    src_even, dst_even, code_even, _ = _pad_edges(
        new_edge_index[0].astype(jnp.int32),
        new_edge_index[1].astype(jnp.int32), codes_even, Mpad - 1)
    src_odd, dst_odd, code_odd, _ = _pad_edges(
        cayley_g[0].astype(jnp.int32), cayley_g[1].astype(jnp.int32),
        jnp.zeros((C,), jnp.int32), Mpad - 1)

    dummy_ee = jnp.zeros((64 * D,), jnp.float32)

    sc_embed, k_emb = _sc_gather_scatter(Npad, emb_src.shape[0],
                                         with_ee=False)
    parts0 = sc_embed(atom_flat, emb_src.reshape(-1, k_emb),
                      emb_dst.reshape(-1, k_emb),
                      emb_code.reshape(-1, k_emb), dummy_ee)

    h, ee_all = _tc_prep(N, M)(parts0, bond_emb)

    sc_even, k_ev = _sc_gather_scatter(Mpad, src_even.shape[0], with_ee=True)
    sc_odd, k_od = _sc_gather_scatter(Mpad, src_odd.shape[0], with_ee=True)
    src_even, dst_even, code_even = (a.reshape(-1, k_ev) for a in
                                     (src_even, dst_even, code_even))
    src_odd, dst_odd, code_odd = (a.reshape(-1, k_od) for a in
                                  (src_odd, dst_odd, code_odd))

    mlp1 = _tc_mlp1(M)
    mlp2 = _tc_mlp2(M)
    for l in range(L):
        eel = ee_all[l].reshape(-1)
        if l % 2 == 1:
            parts = sc_odd(h, src_odd, dst_odd, code_odd, eel)
        else:
            parts = sc_even(h, src_even, dst_even, code_even, eel)
        z1, m1 = mlp1(h, parts, W1[l], b1[l], eps[l])
        v1 = jnp.mean(jnp.square(z1 - m1[0]), axis=0)
        z1d = (z1 - m1[0]) / jnp.sqrt(v1 + 1e-5)
        z2, m2 = mlp2(z1d, g1[l], be1[l], W2[l], b2[l])
        v2 = jnp.mean(jnp.square(z2 - m2[0]), axis=0)
        z2d = (z2 - m2[0]) / jnp.sqrt(v2 + 1e-5)
        h = _tc_bnout(M, last=(l == L - 1))(z2d, bn_g[l], bn_b[l])
    return h


# speedup vs baseline: 4.3398x; 1.0517x over previous
import functools

import jax
import jax.numpy as jnp
from jax import lax
from jax.experimental import pallas as pl
from jax.experimental.pallas import tpu as pltpu
from jax.experimental.pallas import tpu_sc as plsc

D = 128
NC = 2
NS = 16
NW = NC * NS
K = 128


def _round_up(a, b):
    return (a + b - 1) // b * b


SCHUNK = 1024


def _sc_gather_scatter(n_agg, n_edges, with_ee):
    if with_ee:
        K, SUB = 32, 16
    else:
        K, SUB = 128, 8
    SCK = K * SUB
    per_w = n_edges // NW
    n_super = per_w // SCK
    rps = n_agg // NS

    mesh = plsc.VectorSubcoreMesh(core_axis_name="c", subcore_axis_name="s")

    scratch = [
        pltpu.VMEM((SUB, K), jnp.int32),
        pltpu.VMEM((SUB, K), jnp.int32),
        pltpu.VMEM((SUB, K), jnp.int32),
        pltpu.VMEM((2, K, D), jnp.float32),
        pltpu.VMEM((64 * D,), jnp.float32),
        pltpu.VMEM_SHARED((n_agg, D), jnp.float32),
        pltpu.SemaphoreType.DMA,
        pltpu.SemaphoreType.DMA,
    ]
    if not with_ee:
        scratch[2] = pltpu.VMEM((8,), jnp.int32)
        scratch[4] = pltpu.VMEM((16,), jnp.float32)

    def body(table_hbm, src_hbm, dst_hbm, code_hbm, ee_hbm, out_hbm,
             src_m, dst_m, code_m, rows_v, ee_v, agg_sh, sem, sem_s):
        c = lax.axis_index("c")
        s = lax.axis_index("s")
        wid = s * NC + c

        zero16 = jnp.zeros((16,), jnp.float32)
        rv0 = rows_v.at[0]

        def zfill(i, carry):
            for j in range(D // 16):
                rv0[i, pl.ds(j * 16, 16)] = zero16
            return carry

        lax.fori_loop(0, K, zfill, 0)
        row0 = s * rps
        for t in range(rps // K):
            pltpu.sync_copy(rows_v.at[0], agg_sh.at[pl.ds(row0 + t * K, K)])
        rem = rps % K
        if rem:
            pltpu.sync_copy(rows_v.at[0, pl.ds(0, rem)],
                            agg_sh.at[pl.ds(row0 + (rps // K) * K, rem)])
        if with_ee:
            pltpu.sync_copy(ee_hbm, ee_v)
        plsc.subcore_barrier()

        crow0 = wid * (per_w // K)
        lanes = lax.iota(jnp.int32, 16)

        def superchunk(t, carry):
            cbase = crow0 + t * SUB
            i1 = pltpu.async_copy(src_hbm.at[pl.ds(cbase, SUB)], src_m, sem)
            i2 = pltpu.async_copy(dst_hbm.at[pl.ds(cbase, SUB)], dst_m, sem)
            if with_ee:
                i3 = pltpu.async_copy(code_hbm.at[pl.ds(cbase, SUB)],
                                      code_m, sem)
            i1.wait()
            i2.wait()
            if with_ee:
                i3.wait()

            gd = [None] * SUB
            sd = [None] * SUB
            gd[0] = pltpu.async_copy(table_hbm.at[src_m.at[0]],
                                     rows_v.at[0], sem)
            for j in range(SUB):
                b = j % 2
                if j + 1 < SUB:
                    if j >= 1:
                        sd[j - 1].wait()
                    gd[j + 1] = pltpu.async_copy(
                        table_hbm.at[src_m.at[j + 1]],
                        rows_v.at[(j + 1) % 2], sem)
                gd[j].wait()
                if with_ee:
                    rvb = rows_v.at[b]

                    def msg16(g, cr, j=j, rvb=rvb):
                        codes = code_m[j, pl.ds(g * 16, 16)]
                        codesD = codes * D

                        def one(e2, cr2):
                            for u in range(2):
                                e = e2 * 2 + u
                                cbD = jnp.take_along_axis(
                                    codesD, jnp.full((16,), e, jnp.int32),
                                    axis=0)
                                ee_base = cbD + lanes
                                row = g * 16 + e
                                for q in range(D // 16):
                                    sl = pl.ds(q * 16, 16)
                                    eev = plsc.load_gather(
                                        ee_v, [ee_base + q * 16])
                                    rvb[row, sl] = jnp.maximum(
                                        rvb[row, sl] + eev, 0.0)
                            return cr2

                        lax.fori_loop(0, 8, one, 0)
                        return cr

                    lax.fori_loop(0, K // 16, msg16, 0)
                sd[j] = pltpu.async_copy(rows_v.at[b],
                                         agg_sh.at[dst_m.at[j]], sem_s,
                                         add=True)
            sd[SUB - 2].wait()
            sd[SUB - 1].wait()
            return carry

        lax.fori_loop(0, n_super, superchunk, 0)
        plsc.subcore_barrier()

        for t in range(rps // K):
            pltpu.sync_copy(agg_sh.at[pl.ds(row0 + t * K, K)],
                            out_hbm.at[c, pl.ds(row0 + t * K, K)])
        if rem:
            pltpu.sync_copy(agg_sh.at[pl.ds(row0 + (rps // K) * K, rem)],
                            out_hbm.at[c, pl.ds(row0 + (rps // K) * K, rem)])

    return pl.kernel(
        body,
        out_type=jax.ShapeDtypeStruct((NC, n_agg, D), jnp.float32),
        mesh=mesh,
        scratch_types=scratch,
        compiler_params=pltpu.CompilerParams(needs_layout_passes=False),
    ), K


def _tc_prep(N, M):
    def body(p_ref, bond_ref, h_ref, ee_ref):
        h0 = p_ref[0, :N, :] + p_ref[1, :N, :]
        h_ref[:N, :] = h0
        avg = jnp.mean(h0, axis=0, keepdims=True)
        h_ref[N:, :] = jnp.broadcast_to(avg, (M - N, D))
        b0 = bond_ref[:, 0]
        b1 = bond_ref[:, 1]
        b2 = bond_ref[:, 2]
        ee = (b0[:, :, None, None, :] + b1[:, None, :, None, :]
              + b2[:, None, None, :, :])
        Lw = ee.shape[0]
        ee_ref[...] = ee.reshape(Lw, 64, D)

    def run(parts, bond_emb):
        Lw = bond_emb.shape[0]
        return pl.pallas_call(
            body,
            out_shape=(jax.ShapeDtypeStruct((M, D), jnp.float32),
                       jax.ShapeDtypeStruct((Lw, 64, D), jnp.float32)),
        )(parts, bond_emb)

    return run


def _tc_mlp1(M):
    def body(h_ref, p_ref, W1_ref, b1_ref, eps_ref, z1_o, m_o):
        agg = p_ref[0, :M, :] + p_ref[1, :M, :]
        z = (1.0 + eps_ref[0, 0]) * h_ref[...] + agg
        z1 = jnp.dot(z, W1_ref[...], preferred_element_type=jnp.float32)
        z1 = z1 + b1_ref[...]
        z1_o[...] = z1
        m_o[...] = jnp.mean(z1, axis=0, keepdims=True)

    def run(h, parts, W1l, b1l, epsl):
        return pl.pallas_call(
            body,
            out_shape=(jax.ShapeDtypeStruct((M, 2 * D), jnp.float32),
                       jax.ShapeDtypeStruct((1, 2 * D), jnp.float32)),
        )(h, parts, W1l, b1l.reshape(1, -1), epsl.reshape(1, 1))

    return run


def _tc_mlp2(M):
    def body(z1d_ref, g_ref, be_ref, W2_ref, b2_ref, z2_o, m_o):
        z1n = z1d_ref[...] * g_ref[...] + be_ref[...]
        z1r = jnp.maximum(z1n, 0.0)
        z2 = jnp.dot(z1r, W2_ref[...], preferred_element_type=jnp.float32)
        z2 = z2 + b2_ref[...]
        z2_o[...] = z2
        m_o[...] = jnp.mean(z2, axis=0, keepdims=True)

    def run(z1d, g1l, be1l, W2l, b2l):
        return pl.pallas_call(
            body,
            out_shape=(jax.ShapeDtypeStruct((M, D), jnp.float32),
                       jax.ShapeDtypeStruct((1, D), jnp.float32)),
        )(z1d, g1l.reshape(1, -1), be1l.reshape(1, -1), W2l,
          b2l.reshape(1, -1))

    return run


def _tc_bnout(M, last):
    def body(z2d_ref, g_ref, b_ref, o_ref):
        z2n = z2d_ref[...] * g_ref[...] + b_ref[...]
        if not last:
            z2n = jnp.maximum(z2n, 0.0)
        o_ref[...] = z2n

    def run(z2d, gl, bl):
        return pl.pallas_call(
            body, out_shape=jax.ShapeDtypeStruct((M, D), jnp.float32),
        )(z2d, gl.reshape(1, -1), bl.reshape(1, -1))

    return run


def _pad_edges(src, dst, code, trash):
    n = src.shape[0]
    npad = _round_up(n, NW * SCHUNK)
    pad = npad - n
    if pad:
        src = jnp.concatenate([src, jnp.zeros((pad,), jnp.int32)])
        dst = jnp.concatenate([dst, jnp.full((pad,), trash, jnp.int32)])
        code = jnp.concatenate([code, jnp.zeros((pad,), jnp.int32)])
    return src, dst, code, npad


def kernel(x, edge_index, edge_attr, new_edge_index, cayley_g, cayley_attr,
           max_node, atom_emb, bond_emb, W1, b1, g1, be1, W2, b2, eps,
           bn_g, bn_b):
    N = x.shape[0]
    E = edge_attr.shape[0]
    E2 = new_edge_index.shape[1]
    C = cayley_g.shape[1]
    M = E2 - E + N
    L = W1.shape[0]

    Npad = _round_up(N + 1, K)
    Mpad = _round_up(M + 1, K)

    x32 = x.astype(jnp.int32)
    atom_flat = atom_emb.reshape(9 * atom_emb.shape[1], D)
    emb_src = (x32 + jnp.arange(9, dtype=jnp.int32)[None, :]
               * atom_emb.shape[1]).reshape(-1)
    emb_dst = jnp.repeat(jnp.arange(N, dtype=jnp.int32), 9)
    zero_codes = jnp.zeros_like(emb_src)
    emb_src, emb_dst, emb_code, _ = _pad_edges(emb_src, emb_dst, zero_codes,
                                               Npad - 1)

    ea = edge_attr.astype(jnp.int32)
    codes_e = ea[:, 0] * 16 + ea[:, 1] * 4 + ea[:, 2]
    codes_even = jnp.concatenate(
        [codes_e, jnp.zeros((E2 - E,), jnp.int32)])
    src_even, dst_even, code_even, _ = _pad_edges(
        new_edge_index[0].astype(jnp.int32),
        new_edge_index[1].astype(jnp.int32), codes_even, Mpad - 1)
    src_odd, dst_odd, code_odd, _ = _pad_edges(
        cayley_g[0].astype(jnp.int32), cayley_g[1].astype(jnp.int32),
        jnp.zeros((C,), jnp.int32), Mpad - 1)

    dummy_ee = jnp.zeros((64 * D,), jnp.float32)

    sc_embed, k_emb = _sc_gather_scatter(Npad, emb_src.shape[0],
                                         with_ee=False)
    parts0 = sc_embed(atom_flat, emb_src.reshape(-1, k_emb),
                      emb_dst.reshape(-1, k_emb),
                      emb_code.reshape(-1, k_emb), dummy_ee)

    h, ee_all = _tc_prep(N, M)(parts0, bond_emb)

    sc_even, k_ev = _sc_gather_scatter(Mpad, src_even.shape[0], with_ee=True)
    sc_odd, k_od = _sc_gather_scatter(Mpad, src_odd.shape[0], with_ee=True)
    src_even, dst_even, code_even = (a.reshape(-1, k_ev) for a in
                                     (src_even, dst_even, code_even))
    src_odd, dst_odd, code_odd = (a.reshape(-1, k_od) for a in
                                  (src_odd, dst_odd, code_odd))

    mlp1 = _tc_mlp1(M)
    mlp2 = _tc_mlp2(M)
    for l in range(L):
        eel = ee_all[l].reshape(-1)
        if l % 2 == 1:
            parts = sc_odd(h, src_odd, dst_odd, code_odd, eel)
        else:
            parts = sc_even(h, src_even, dst_even, code_even, eel)
        z1, m1 = mlp1(h, parts, W1[l], b1[l], eps[l])
        v1 = jnp.mean(jnp.square(z1 - m1[0]), axis=0)
        z1d = (z1 - m1[0]) / jnp.sqrt(v1 + 1e-5)
        z2, m2 = mlp2(z1d, g1[l], be1[l], W2[l], b2[l])
        v2 = jnp.mean(jnp.square(z2 - m2[0]), axis=0)
        z2d = (z2 - m2[0]) / jnp.sqrt(v2 + 1e-5)
        h = _tc_bnout(M, last=(l == L - 1))(z2d, bn_g[l], bn_b[l])
    return h


# speedup vs baseline: 4.3445x; 1.0011x over previous
import functools

import jax
import jax.numpy as jnp
from jax import lax
from jax.experimental import pallas as pl
from jax.experimental.pallas import tpu as pltpu
from jax.experimental.pallas import tpu_sc as plsc

D = 128
NC = 2
NS = 16
NW = NC * NS
K = 128


def _round_up(a, b):
    return (a + b - 1) // b * b


SCHUNK = 1024


def _sc_gather_scatter(n_agg, n_edges, with_ee):
    if with_ee:
        K, SUB = 32, 16
    else:
        K, SUB = 128, 8
    SCK = K * SUB
    per_w = n_edges // NW
    n_super = per_w // SCK
    rps = n_agg // NS

    mesh = plsc.VectorSubcoreMesh(core_axis_name="c", subcore_axis_name="s")

    scratch = [
        pltpu.VMEM((SUB, K), jnp.int32),
        pltpu.VMEM((SUB, K), jnp.int32),
        pltpu.VMEM((SUB, K), jnp.int32),
        pltpu.VMEM((2, K, D), jnp.float32),
        pltpu.VMEM((64 * D,), jnp.float32),
        pltpu.VMEM_SHARED((n_agg, D), jnp.float32),
        pltpu.SemaphoreType.DMA,
        pltpu.SemaphoreType.DMA,
    ]
    if not with_ee:
        scratch[2] = pltpu.VMEM((8,), jnp.int32)
        scratch[4] = pltpu.VMEM((16,), jnp.float32)

    def body(table_hbm, src_hbm, dst_hbm, code_hbm, ee_hbm, out_hbm,
             src_m, dst_m, code_m, rows_v, ee_v, agg_sh, sem, sem_s):
        c = lax.axis_index("c")
        s = lax.axis_index("s")
        wid = s * NC + c

        zero16 = jnp.zeros((16,), jnp.float32)
        rv0 = rows_v.at[0]

        def zfill(i, carry):
            for j in range(D // 16):
                rv0[i, pl.ds(j * 16, 16)] = zero16
            return carry

        lax.fori_loop(0, K, zfill, 0)
        row0 = s * rps
        for t in range(rps // K):
            pltpu.sync_copy(rows_v.at[0], agg_sh.at[pl.ds(row0 + t * K, K)])
        rem = rps % K
        if rem:
            pltpu.sync_copy(rows_v.at[0, pl.ds(0, rem)],
                            agg_sh.at[pl.ds(row0 + (rps // K) * K, rem)])
        if with_ee:
            pltpu.sync_copy(ee_hbm, ee_v)
        plsc.subcore_barrier()

        crow0 = wid * (per_w // K)
        lanes = lax.iota(jnp.int32, 16)

        def superchunk(t, carry):
            cbase = crow0 + t * SUB
            i1 = pltpu.async_copy(src_hbm.at[pl.ds(cbase, SUB)], src_m, sem)
            i2 = pltpu.async_copy(dst_hbm.at[pl.ds(cbase, SUB)], dst_m, sem)
            if with_ee:
                i3 = pltpu.async_copy(code_hbm.at[pl.ds(cbase, SUB)],
                                      code_m, sem)
            i1.wait()
            i2.wait()
            if with_ee:
                i3.wait()

            gd = [None] * SUB
            sd = [None] * SUB
            gd[0] = pltpu.async_copy(table_hbm.at[src_m.at[0]],
                                     rows_v.at[0], sem)
            for j in range(SUB):
                b = j % 2
                if j + 1 < SUB:
                    if j >= 1:
                        sd[j - 1].wait()
                    gd[j + 1] = pltpu.async_copy(
                        table_hbm.at[src_m.at[j + 1]],
                        rows_v.at[(j + 1) % 2], sem)
                gd[j].wait()
                if with_ee:
                    rvb = rows_v.at[b]

                    def msg16(g, cr, j=j, rvb=rvb):
                        codes = code_m[j, pl.ds(g * 16, 16)]
                        codesD = codes * D

                        def one(e2, cr2):
                            for u in range(2):
                                e = e2 * 2 + u
                                cbD = jnp.take_along_axis(
                                    codesD, jnp.full((16,), e, jnp.int32),
                                    axis=0)
                                ee_base = cbD + lanes
                                row = g * 16 + e
                                for q in range(D // 16):
                                    sl = pl.ds(q * 16, 16)
                                    eev = plsc.load_gather(
                                        ee_v, [ee_base + q * 16])
                                    rvb[row, sl] = jnp.maximum(
                                        rvb[row, sl] + eev, 0.0)
                            return cr2

                        lax.fori_loop(0, 8, one, 0)
                        return cr

                    lax.fori_loop(0, K // 16, msg16, 0)
                sd[j] = pltpu.async_copy(rows_v.at[b],
                                         agg_sh.at[dst_m.at[j]], sem_s,
                                         add=True)
            sd[SUB - 2].wait()
            sd[SUB - 1].wait()
            return carry

        lax.fori_loop(0, n_super, superchunk, 0)
        plsc.subcore_barrier()

        for t in range(rps // K):
            pltpu.sync_copy(agg_sh.at[pl.ds(row0 + t * K, K)],
                            out_hbm.at[c, pl.ds(row0 + t * K, K)])
        if rem:
            pltpu.sync_copy(agg_sh.at[pl.ds(row0 + (rps // K) * K, rem)],
                            out_hbm.at[c, pl.ds(row0 + (rps // K) * K, rem)])

    return pl.kernel(
        body,
        out_type=jax.ShapeDtypeStruct((NC, n_agg, D), jnp.float32),
        mesh=mesh,
        scratch_types=scratch,
        compiler_params=pltpu.CompilerParams(needs_layout_passes=False),
    ), K


def _tc_prep(N, M):
    def body(p_ref, bond_ref, h_ref, ee_ref):
        h0 = p_ref[0, :N, :] + p_ref[1, :N, :]
        h_ref[:N, :] = h0
        avg = jnp.mean(h0, axis=0, keepdims=True)
        h_ref[N:, :] = jnp.broadcast_to(avg, (M - N, D))
        b0 = bond_ref[:, 0]
        b1 = bond_ref[:, 1]
        b2 = bond_ref[:, 2]
        ee = (b0[:, :, None, None, :] + b1[:, None, :, None, :]
              + b2[:, None, None, :, :])
        Lw = ee.shape[0]
        ee_ref[...] = ee.reshape(Lw, 64, D)

    def run(parts, bond_emb):
        Lw = bond_emb.shape[0]
        return pl.pallas_call(
            body,
            out_shape=(jax.ShapeDtypeStruct((M, D), jnp.float32),
                       jax.ShapeDtypeStruct((Lw, 64, D), jnp.float32)),
        )(parts, bond_emb)

    return run


def _tc_mlp1(M):
    def body(h_ref, p_ref, W1_ref, b1_ref, eps_ref, z1_o, m_o):
        agg = p_ref[0, :M, :] + p_ref[1, :M, :]
        z = (1.0 + eps_ref[0, 0]) * h_ref[...] + agg
        z1 = jnp.dot(z, W1_ref[...], preferred_element_type=jnp.float32)
        z1 = z1 + b1_ref[...]
        z1_o[...] = z1
        m_o[...] = jnp.mean(z1, axis=0, keepdims=True)

    def run(h, parts, W1l, b1l, epsl):
        return pl.pallas_call(
            body,
            out_shape=(jax.ShapeDtypeStruct((M, 2 * D), jnp.float32),
                       jax.ShapeDtypeStruct((1, 2 * D), jnp.float32)),
        )(h, parts, W1l, b1l.reshape(1, -1), epsl.reshape(1, 1))

    return run


def _tc_mlp2(M):
    def body(z1d_ref, g_ref, be_ref, W2_ref, b2_ref, z2_o, m_o):
        z1n = z1d_ref[...] * g_ref[...] + be_ref[...]
        z1r = jnp.maximum(z1n, 0.0)
        z2 = jnp.dot(z1r, W2_ref[...], preferred_element_type=jnp.float32)
        z2 = z2 + b2_ref[...]
        z2_o[...] = z2
        m_o[...] = jnp.mean(z2, axis=0, keepdims=True)

    def run(z1d, g1l, be1l, W2l, b2l):
        return pl.pallas_call(
            body,
            out_shape=(jax.ShapeDtypeStruct((M, D), jnp.float32),
                       jax.ShapeDtypeStruct((1, D), jnp.float32)),
        )(z1d, g1l.reshape(1, -1), be1l.reshape(1, -1), W2l,
          b2l.reshape(1, -1))

    return run


def _tc_bnout(M, last):
    def body(z2d_ref, g_ref, b_ref, o_ref):
        z2n = z2d_ref[...] * g_ref[...] + b_ref[...]
        if not last:
            z2n = jnp.maximum(z2n, 0.0)
        o_ref[...] = z2n

    def run(z2d, gl, bl):
        return pl.pallas_call(
            body, out_shape=jax.ShapeDtypeStruct((M, D), jnp.float32),
        )(z2d, gl.reshape(1, -1), bl.reshape(1, -1))

    return run


def _pad_edges(src, dst, code, trash):
    n = src.shape[0]
    npad = _round_up(n, NW * SCHUNK)
    pad = npad - n
    if pad:
        src = jnp.concatenate([src, jnp.zeros((pad,), jnp.int32)])
        dst = jnp.concatenate([dst, jnp.full((pad,), trash, jnp.int32)])
        code = jnp.concatenate([code, jnp.zeros((pad,), jnp.int32)])
    return src, dst, code, npad


def kernel(x, edge_index, edge_attr, new_edge_index, cayley_g, cayley_attr,
           max_node, atom_emb, bond_emb, W1, b1, g1, be1, W2, b2, eps,
           bn_g, bn_b):
    N = x.shape[0]
    E = edge_attr.shape[0]
    E2 = new_edge_index.shape[1]
    C = cayley_g.shape[1]
    M = E2 - E + N
    L = W1.shape[0]

    Npad = _round_up(N + 1, K)
    Mpad = _round_up(M + 1, K)

    x32 = x.astype(jnp.int32)
    atom_flat = atom_emb.reshape(9 * atom_emb.shape[1], D)
    emb_src = (x32 + jnp.arange(9, dtype=jnp.int32)[None, :]
               * atom_emb.shape[1]).T.reshape(-1)
    emb_dst = jnp.tile(jnp.arange(N, dtype=jnp.int32), 9)
    zero_codes = jnp.zeros_like(emb_src)
    emb_src, emb_dst, emb_code, _ = _pad_edges(emb_src, emb_dst, zero_codes,
                                               Npad - 1)

    ea = edge_attr.astype(jnp.int32)
    codes_e = ea[:, 0] * 16 + ea[:, 1] * 4 + ea[:, 2]
    codes_even = jnp.concatenate(
        [codes_e, jnp.zeros((E2 - E,), jnp.int32)])
    src_even, dst_even, code_even, _ = _pad_edges(
        new_edge_index[0].astype(jnp.int32),
        new_edge_index[1].astype(jnp.int32), codes_even, Mpad - 1)
    src_odd, dst_odd, code_odd, _ = _pad_edges(
        cayley_g[0].astype(jnp.int32), cayley_g[1].astype(jnp.int32),
        jnp.zeros((C,), jnp.int32), Mpad - 1)

    dummy_ee = jnp.zeros((64 * D,), jnp.float32)

    sc_embed, k_emb = _sc_gather_scatter(Npad, emb_src.shape[0],
                                         with_ee=False)
    parts0 = sc_embed(atom_flat, emb_src.reshape(-1, k_emb),
                      emb_dst.reshape(-1, k_emb),
                      emb_code.reshape(-1, k_emb), dummy_ee)

    h, ee_all = _tc_prep(N, M)(parts0, bond_emb)

    sc_even, k_ev = _sc_gather_scatter(Mpad, src_even.shape[0], with_ee=True)
    sc_odd, k_od = _sc_gather_scatter(Mpad, src_odd.shape[0], with_ee=True)
    src_even, dst_even, code_even = (a.reshape(-1, k_ev) for a in
                                     (src_even, dst_even, code_even))
    src_odd, dst_odd, code_odd = (a.reshape(-1, k_od) for a in
                                  (src_odd, dst_odd, code_odd))

    mlp1 = _tc_mlp1(M)
    mlp2 = _tc_mlp2(M)
    for l in range(L):
        eel = ee_all[l].reshape(-1)
        if l % 2 == 1:
            parts = sc_odd(h, src_odd, dst_odd, code_odd, eel)
        else:
            parts = sc_even(h, src_even, dst_even, code_even, eel)
        z1, m1 = mlp1(h, parts, W1[l], b1[l], eps[l])
        v1 = jnp.mean(jnp.square(z1 - m1[0]), axis=0)
        z1d = (z1 - m1[0]) / jnp.sqrt(v1 + 1e-5)
        z2, m2 = mlp2(z1d, g1[l], be1[l], W2[l], b2[l])
        v2 = jnp.mean(jnp.square(z2 - m2[0]), axis=0)
        z2d = (z2 - m2[0]) / jnp.sqrt(v2 + 1e-5)
        h = _tc_bnout(M, last=(l == L - 1))(z2d, bn_g[l], bn_b[l])
    return h
